# Initial kernel scaffold; baseline (speedup 1.0000x reference)
import functools

import jax
import jax.numpy as jnp
from jax import lax
from jax.experimental import pallas as pl
from jax.experimental.pallas import tpu as pltpu
from jax.experimental.pallas import tpu_sc as plsc

_BN_EPS = 1e-5


def _mm_bias(x, w, b, block_rows):
    m, k = x.shape
    _, n = w.shape
    assert m % block_rows == 0

    def body(x_ref, w_ref, b_ref, o_ref):
        o_ref[...] = (
            jnp.dot(x_ref[...], w_ref[...], preferred_element_type=jnp.float32)
            + b_ref[...]
        )

    return pl.pallas_call(
        body,
        grid=(m // block_rows,),
        in_specs=[
            pl.BlockSpec((block_rows, k), lambda i: (i, 0)),
            pl.BlockSpec((k, n), lambda i: (0, 0)),
            pl.BlockSpec((1, n), lambda i: (0, 0)),
        ],
        out_specs=pl.BlockSpec((block_rows, n), lambda i: (i, 0)),
        out_shape=jax.ShapeDtypeStruct((m, n), jnp.float32),
    )(x, w, b.reshape(1, n))


def _layer_dense(h, agg2, lp):
    n_nodes, hdim = h.shape

    def body(h_ref, a_ref, eps_ref, w1_ref, b1_ref, g1_ref, be1_ref,
             w2_ref, b2_ref, gn_ref, bn_ref, o_ref):
        a = a_ref[0:n_nodes, :] + a_ref[n_nodes:2 * n_nodes, :]
        z = (1.0 + eps_ref[0, 0]) * h_ref[...] + a
        t = jnp.dot(z, w1_ref[...], preferred_element_type=jnp.float32) + b1_ref[...]
        m = jnp.mean(t, axis=0, keepdims=True)
        v = jnp.mean((t - m) * (t - m), axis=0, keepdims=True)
        t = g1_ref[...] * (t - m) * lax.rsqrt(v + _BN_EPS) + be1_ref[...]
        t = jnp.maximum(t, 0.0)
        u = jnp.dot(t, w2_ref[...], preferred_element_type=jnp.float32) + b2_ref[...]
        m2 = jnp.mean(u, axis=0, keepdims=True)
        v2 = jnp.mean((u - m2) * (u - m2), axis=0, keepdims=True)
        u = gn_ref[...] * (u - m2) * lax.rsqrt(v2 + _BN_EPS) + bn_ref[...]
        o_ref[...] = jnp.maximum(u, 0.0)

    h2 = lp["W1"].shape[1]
    return pl.pallas_call(
        body,
        out_shape=jax.ShapeDtypeStruct((n_nodes, hdim), jnp.float32),
    )(
        h,
        agg2,
        lp["eps"].reshape(1, 1),
        lp["W1"],
        lp["b1"].reshape(1, h2),
        lp["g1"].reshape(1, h2),
        lp["be1"].reshape(1, h2),
        lp["W2"],
        lp["b2"].reshape(1, hdim),
        lp["gn"].reshape(1, hdim),
        lp["bn"].reshape(1, hdim),
    )


def _make_aggregate(n_nodes, n_edges, feat):
    info = plsc.get_sparse_core_info()
    nc, ns = info.num_cores, info.num_subcores
    nw = nc * ns
    e_per_w = n_edges // nw
    assert e_per_w * nw == n_edges
    GB = 80
    KG = 5
    B = GB * KG
    n_chunks = e_per_w // B
    assert n_chunks * B == e_per_w
    rows_pt = n_nodes // ns
    assert rows_pt * ns == n_nodes
    zr = 125
    nz = rows_pt // zr
    assert nz * zr == rows_pt

    mesh = plsc.VectorSubcoreMesh(core_axis_name="c", subcore_axis_name="s")

    @functools.partial(
        pl.kernel,
        out_type=jax.ShapeDtypeStruct((nc * n_nodes, feat), jnp.float32),
        mesh=mesh,
        scratch_types=[
            pltpu.VMEM((KG, GB), jnp.int32),
            pltpu.VMEM((KG, GB), jnp.int32),
            pltpu.VMEM((B, feat), jnp.float32),
            pltpu.VMEM((B, feat), jnp.float32),
            pltpu.VMEM_SHARED((n_nodes, feat), jnp.float32),
            pltpu.SemaphoreType.DMA,
            pltpu.SemaphoreType.DMA,
        ],
    )
    def k(h_hbm, ea_hbm, src_hbm, dst_hbm, out_hbm,
          srcv, dstv, hrows, eav, aggr_sh, sem_g, sem_e):
        c = lax.axis_index("c")
        s = lax.axis_index("s")
        wid = c * ns + s

        def zrow(r, _):
            for f in range(feat // 16):
                hrows[r, pl.ds(16 * f, 16)] = jnp.zeros((16,), jnp.float32)
            return 0
        lax.fori_loop(0, zr, zrow, 0)
        for j in range(nz):
            pltpu.sync_copy(
                hrows.at[pl.ds(0, zr)],
                aggr_sh.at[pl.ds(s * rows_pt + j * zr, zr)],
            )
        plsc.subcore_barrier()

        def chunk_body(i, _):
            base = wid * e_per_w + i * B
            pltpu.sync_copy(src_hbm.at[pl.ds(base, B)], srcv)
            pltpu.sync_copy(dst_hbm.at[pl.ds(base, B)], dstv)
            cp_e = pltpu.async_copy(ea_hbm.at[pl.ds(base, B)], eav, sem_e)
            gathers = []
            for j in range(KG):
                gathers.append(
                    pltpu.async_copy(
                        h_hbm.at[srcv.at[j]],
                        hrows.at[pl.ds(j * GB, GB)],
                        sem_g,
                    )
                )
            cp_e.wait()
            for g in gathers:
                g.wait()

            def row_body(r, _):
                for f in range(feat // 16):
                    sl = pl.ds(16 * f, 16)
                    eav[r, sl] = jnp.maximum(eav[r, sl] + hrows[r, sl], 0.0)
                return 0
            lax.fori_loop(0, B, row_body, 0)

            for j in range(KG):
                pltpu.sync_copy(
                    eav.at[pl.ds(j * GB, GB)],
                    aggr_sh.at[dstv.at[j]],
                    add=True,
                )
            return 0
        lax.fori_loop(0, n_chunks, chunk_body, 0)

        plsc.subcore_barrier()

        pltpu.sync_copy(
            aggr_sh.at[pl.ds(s * rows_pt, rows_pt)],
            out_hbm.at[pl.ds(c * n_nodes + s * rows_pt, rows_pt)],
        )

    return k


def kernel(x, edge_attr, params, edge_index):
    n_nodes, _ = x.shape
    n_edges = edge_attr.shape[0]
    h = _mm_bias(x, params["atom_W"], params["atom_b"], 2000)
    ea = _mm_bias(edge_attr, params["bond_W"], params["bond_b"], 2000)
    feat = h.shape[1]
    src = edge_index[0]
    dst = edge_index[1]
    agg_fn = _make_aggregate(n_nodes, n_edges, feat)
    for lp in params["layers"]:
        agg2 = agg_fn(h, ea, src, dst)
        h = _layer_dense(h, agg2, lp)
    return _mm_bias(h, params["lin_W"], params["lin_b"], 2000)


# speedup vs baseline: 2.5449x; 2.5449x over previous
import functools

import jax
import jax.numpy as jnp
from jax import lax
from jax.experimental import pallas as pl
from jax.experimental.pallas import tpu as pltpu
from jax.experimental.pallas import tpu_sc as plsc

_BN_EPS = 1e-5


def _mm_bias(x, w, b, block_rows):
    m, k = x.shape
    _, n = w.shape
    assert m % block_rows == 0

    def body(x_ref, w_ref, b_ref, o_ref):
        o_ref[...] = (
            jnp.dot(x_ref[...], w_ref[...], preferred_element_type=jnp.float32)
            + b_ref[...]
        )

    return pl.pallas_call(
        body,
        grid=(m // block_rows,),
        in_specs=[
            pl.BlockSpec((block_rows, k), lambda i: (i, 0)),
            pl.BlockSpec((k, n), lambda i: (0, 0)),
            pl.BlockSpec((1, n), lambda i: (0, 0)),
        ],
        out_specs=pl.BlockSpec((block_rows, n), lambda i: (i, 0)),
        out_shape=jax.ShapeDtypeStruct((m, n), jnp.float32),
    )(x, w, b.reshape(1, n))


def _layer_dense(h, agg2, lp):
    n_nodes, hdim = h.shape

    def body(h_ref, a_ref, eps_ref, w1_ref, b1_ref, g1_ref, be1_ref,
             w2_ref, b2_ref, gn_ref, bn_ref, o_ref):
        a = jnp.concatenate(
            [a_ref[0:n_nodes, :], a_ref[n_nodes:2 * n_nodes, :]], axis=1)
        z = (1.0 + eps_ref[0, 0]) * h_ref[...] + a
        t = jnp.dot(z, w1_ref[...], preferred_element_type=jnp.float32) + b1_ref[...]
        m = jnp.mean(t, axis=0, keepdims=True)
        v = jnp.mean((t - m) * (t - m), axis=0, keepdims=True)
        t = g1_ref[...] * (t - m) * lax.rsqrt(v + _BN_EPS) + be1_ref[...]
        t = jnp.maximum(t, 0.0)
        u = jnp.dot(t, w2_ref[...], preferred_element_type=jnp.float32) + b2_ref[...]
        m2 = jnp.mean(u, axis=0, keepdims=True)
        v2 = jnp.mean((u - m2) * (u - m2), axis=0, keepdims=True)
        u = gn_ref[...] * (u - m2) * lax.rsqrt(v2 + _BN_EPS) + bn_ref[...]
        o_ref[...] = jnp.maximum(u, 0.0)

    h2 = lp["W1"].shape[1]
    return pl.pallas_call(
        body,
        out_shape=jax.ShapeDtypeStruct((n_nodes, hdim), jnp.float32),
    )(
        h,
        agg2,
        lp["eps"].reshape(1, 1),
        lp["W1"],
        lp["b1"].reshape(1, h2),
        lp["g1"].reshape(1, h2),
        lp["be1"].reshape(1, h2),
        lp["W2"],
        lp["b2"].reshape(1, hdim),
        lp["gn"].reshape(1, hdim),
        lp["bn"].reshape(1, hdim),
    )


def _make_aggregate(n_nodes, n_edges, feat):
    info = plsc.get_sparse_core_info()
    nc, ns = info.num_cores, info.num_subcores
    fh = feat // nc
    e_per_t = n_edges // ns
    assert e_per_t * ns == n_edges
    GB = 80
    KG = 5
    B = GB * KG
    n_chunks = e_per_t // B
    assert n_chunks * B == e_per_t
    rows_pt = 2000
    n_out_tiles = n_nodes // rows_pt
    assert n_out_tiles * rows_pt == n_nodes and n_out_tiles <= ns
    zr = 125
    nz = rows_pt // zr
    assert nz * zr == rows_pt

    mesh = plsc.VectorSubcoreMesh(core_axis_name="c", subcore_axis_name="s")

    @functools.partial(
        pl.kernel,
        out_type=jax.ShapeDtypeStruct((nc * n_nodes, fh), jnp.float32),
        mesh=mesh,
        scratch_types=[
            pltpu.VMEM((KG, GB), jnp.int32),
            pltpu.VMEM((KG, GB), jnp.int32),
            pltpu.VMEM((B, fh), jnp.float32),
            pltpu.VMEM((B, fh), jnp.float32),
            pltpu.VMEM_SHARED((n_nodes, fh), jnp.float32),
            pltpu.SemaphoreType.DMA,
            pltpu.SemaphoreType.DMA,
            pltpu.SemaphoreType.DMA,
        ],
        compiler_params=pltpu.CompilerParams(use_tc_tiling_on_sc=False),
    )
    def k(h_hbm, ea_hbm, src_hbm, dst_hbm, out_hbm,
          srcv, dstv, hrows, eav, aggr_sh, sem_g, sem_e, sem_i):
        c = lax.axis_index("c")
        s = lax.axis_index("s")

        @pl.when(s < n_out_tiles)
        def _zero():
            def zrow(r, _):
                for f in range(fh // 16):
                    hrows[r, pl.ds(16 * f, 16)] = jnp.zeros((16,), jnp.float32)
                return 0
            lax.fori_loop(0, zr, zrow, 0)
            for j in range(nz):
                pltpu.sync_copy(
                    hrows.at[pl.ds(0, zr)],
                    aggr_sh.at[pl.ds(s * rows_pt + j * zr, zr)],
                )
        plsc.subcore_barrier()

        def chunk_body(i, _):
            base = s * e_per_t + i * B
            idx_cps = []
            for j in range(KG):
                idx_cps.append(pltpu.async_copy(
                    src_hbm.at[pl.ds(base + j * GB, GB)], srcv.at[j], sem_i))
                idx_cps.append(pltpu.async_copy(
                    dst_hbm.at[pl.ds(base + j * GB, GB)], dstv.at[j], sem_i))
            cp_e = pltpu.async_copy(
                ea_hbm.at[pl.ds(c * n_edges + base, B)], eav, sem_e)
            for cp in idx_cps:
                cp.wait()
            coff = c * n_nodes
            for j in range(KG):
                for t in range(GB // 16):
                    sl = pl.ds(16 * t, 16)
                    srcv[j, sl] = srcv[j, sl] + coff
            gathers = []
            for j in range(KG):
                gathers.append(
                    pltpu.async_copy(
                        h_hbm.at[srcv.at[j]],
                        hrows.at[pl.ds(j * GB, GB)],
                        sem_g,
                    )
                )
            cp_e.wait()
            for g in gathers:
                g.wait()

            def row_body(r, _):
                for f in range(fh // 16):
                    sl = pl.ds(16 * f, 16)
                    eav[r, sl] = jnp.maximum(eav[r, sl] + hrows[r, sl], 0.0)
                return 0
            lax.fori_loop(0, B, row_body, 0)

            for j in range(KG):
                pltpu.sync_copy(
                    eav.at[pl.ds(j * GB, GB)],
                    aggr_sh.at[dstv.at[j]],
                    add=True,
                )
            return 0
        lax.fori_loop(0, n_chunks, chunk_body, 0)

        plsc.subcore_barrier()

        @pl.when(s < n_out_tiles)
        def _writeout():
            pltpu.sync_copy(
                aggr_sh.at[pl.ds(s * rows_pt, rows_pt)],
                out_hbm.at[pl.ds(c * n_nodes + s * rows_pt, rows_pt)],
            )

    return k


def _split(a):
    n = a.shape[1] // 2
    return jnp.concatenate([a[:, :n], a[:, n:]], axis=0)


def kernel(x, edge_attr, params, edge_index):
    n_nodes, _ = x.shape
    n_edges = edge_attr.shape[0]
    h = _mm_bias(x, params["atom_W"], params["atom_b"], 2000)
    ea = _mm_bias(edge_attr, params["bond_W"], params["bond_b"], 2000)
    feat = h.shape[1]
    src = edge_index[0]
    dst = edge_index[1]
    ea_s = _split(ea)
    agg_fn = _make_aggregate(n_nodes, n_edges, feat)
    for lp in params["layers"]:
        agg2 = agg_fn(_split(h), ea_s, src, dst)
        h = _layer_dense(h, agg2, lp)
    return _mm_bias(h, params["lin_W"], params["lin_b"], 2000)


# speedup vs baseline: 4.1204x; 1.6190x over previous
import functools

import jax
import jax.numpy as jnp
from jax import lax
from jax.experimental import pallas as pl
from jax.experimental.pallas import tpu as pltpu
from jax.experimental.pallas import tpu_sc as plsc

_BN_EPS = 1e-5


def _mm_bias(x, w, b, block_rows):
    m, k = x.shape
    _, n = w.shape
    assert m % block_rows == 0

    def body(x_ref, w_ref, b_ref, o_ref):
        o_ref[...] = (
            jnp.dot(x_ref[...], w_ref[...], preferred_element_type=jnp.float32)
            + b_ref[...]
        )

    return pl.pallas_call(
        body,
        grid=(m // block_rows,),
        in_specs=[
            pl.BlockSpec((block_rows, k), lambda i: (i, 0)),
            pl.BlockSpec((k, n), lambda i: (0, 0)),
            pl.BlockSpec((1, n), lambda i: (0, 0)),
        ],
        out_specs=pl.BlockSpec((block_rows, n), lambda i: (i, 0)),
        out_shape=jax.ShapeDtypeStruct((m, n), jnp.float32),
    )(x, w, b.reshape(1, n))


def _layer_dense(h, agg2, lp):
    n_nodes, hdim = h.shape

    def body(h_ref, a_ref, eps_ref, w1_ref, b1_ref, g1_ref, be1_ref,
             w2_ref, b2_ref, gn_ref, bn_ref, o_ref):
        a = a_ref[0:n_nodes, :] + a_ref[n_nodes:2 * n_nodes, :]
        z = (1.0 + eps_ref[0, 0]) * h_ref[...] + a
        t = jnp.dot(z, w1_ref[...], preferred_element_type=jnp.float32) + b1_ref[...]
        m = jnp.mean(t, axis=0, keepdims=True)
        v = jnp.mean((t - m) * (t - m), axis=0, keepdims=True)
        t = g1_ref[...] * (t - m) / jnp.sqrt(v + _BN_EPS) + be1_ref[...]
        t = jnp.maximum(t, 0.0)
        u = jnp.dot(t, w2_ref[...], preferred_element_type=jnp.float32) + b2_ref[...]
        m2 = jnp.mean(u, axis=0, keepdims=True)
        v2 = jnp.mean((u - m2) * (u - m2), axis=0, keepdims=True)
        u = gn_ref[...] * (u - m2) / jnp.sqrt(v2 + _BN_EPS) + bn_ref[...]
        o_ref[...] = jnp.maximum(u, 0.0)

    h2 = lp["W1"].shape[1]
    return pl.pallas_call(
        body,
        out_shape=jax.ShapeDtypeStruct((n_nodes, hdim), jnp.float32),
    )(
        h,
        agg2,
        lp["eps"].reshape(1, 1),
        lp["W1"],
        lp["b1"].reshape(1, h2),
        lp["g1"].reshape(1, h2),
        lp["be1"].reshape(1, h2),
        lp["W2"],
        lp["b2"].reshape(1, hdim),
        lp["gn"].reshape(1, hdim),
        lp["bn"].reshape(1, hdim),
    )


def _make_aggregate(n_nodes, n_edges, feat):
    info = plsc.get_sparse_core_info()
    nc, ns = info.num_cores, info.num_subcores
    nw = nc * ns
    B = 128
    base_chunks = n_edges // (B * nw)
    extra = n_edges // B - base_chunks * nw
    e_per_w = base_chunks * B
    assert (base_chunks * nw + extra) * B == n_edges
    rows_pt = 2000
    n_out_tiles = n_nodes // rows_pt
    assert n_out_tiles * rows_pt == n_nodes and n_out_tiles <= ns
    zr = 125
    nz = rows_pt // zr

    mesh = plsc.VectorSubcoreMesh(core_axis_name="c", subcore_axis_name="s")

    @functools.partial(
        pl.kernel,
        out_type=jax.ShapeDtypeStruct((nc * n_nodes, feat), jnp.float32),
        mesh=mesh,
        scratch_types=[
            pltpu.VMEM((1, B), jnp.int32),
            pltpu.VMEM((1, B), jnp.int32),
            pltpu.VMEM((B, feat), jnp.float32),
            pltpu.VMEM((B, feat), jnp.float32),
            pltpu.VMEM_SHARED((n_nodes, feat), jnp.float32),
            pltpu.SemaphoreType.DMA,
            pltpu.SemaphoreType.DMA,
            pltpu.SemaphoreType.DMA,
        ],
    )
    def k(h_hbm, ea_hbm, src_hbm, dst_hbm, out_hbm,
          srcv, dstv, hrows, eav, aggr_sh, sem_g, sem_e, sem_i):
        c = lax.axis_index("c")
        s = lax.axis_index("s")
        wid = c * ns + s

        @pl.when(s < n_out_tiles)
        def _zero():
            def zrow(r, _):
                for f in range(feat // 16):
                    hrows[r, pl.ds(16 * f, 16)] = jnp.zeros((16,), jnp.float32)
                return 0
            lax.fori_loop(0, zr, zrow, 0)
            for j in range(nz):
                pltpu.sync_copy(
                    hrows.at[pl.ds(0, zr)],
                    aggr_sh.at[pl.ds(s * rows_pt + j * zr, zr)],
                )
        plsc.subcore_barrier()

        n_chunks = base_chunks + extra * jnp.int32(wid == nw - 1)

        def chunk_body(i, _):
            base = wid * e_per_w + i * B
            cp_s = pltpu.async_copy(src_hbm.at[pl.ds(base, B)], srcv.at[0], sem_i)
            cp_d = pltpu.async_copy(dst_hbm.at[pl.ds(base, B)], dstv.at[0], sem_i)
            cp_e = pltpu.async_copy(ea_hbm.at[pl.ds(base, B)], eav, sem_e)
            cp_s.wait()
            cp_d.wait()
            cp_g = pltpu.async_copy(h_hbm.at[srcv.at[0]], hrows, sem_g)
            cp_e.wait()
            cp_g.wait()

            def row_body(r, _):
                for f in range(feat // 16):
                    sl = pl.ds(16 * f, 16)
                    eav[r, sl] = jnp.maximum(eav[r, sl] + hrows[r, sl], 0.0)
                return 0
            lax.fori_loop(0, B, row_body, 0)

            pltpu.sync_copy(eav, aggr_sh.at[dstv.at[0]], add=True)
            return 0
        lax.fori_loop(0, n_chunks, chunk_body, 0)

        plsc.subcore_barrier()

        @pl.when(s < n_out_tiles)
        def _writeout():
            pltpu.sync_copy(
                aggr_sh.at[pl.ds(s * rows_pt, rows_pt)],
                out_hbm.at[pl.ds(c * n_nodes + s * rows_pt, rows_pt)],
            )

    return k


def kernel(x, edge_attr, params, edge_index):
    n_nodes, _ = x.shape
    n_edges = edge_attr.shape[0]
    h = _mm_bias(x, params["atom_W"], params["atom_b"], 2000)
    ea = _mm_bias(edge_attr, params["bond_W"], params["bond_b"], 2000)
    feat = h.shape[1]
    src = edge_index[0]
    dst = edge_index[1]
    agg_fn = _make_aggregate(n_nodes, n_edges, feat)
    for lp in params["layers"]:
        agg2 = agg_fn(h, ea, src, dst)
        h = _layer_dense(h, agg2, lp)
    return _mm_bias(h, params["lin_W"], params["lin_b"], 2000)
